# Initial kernel scaffold; baseline (speedup 1.0000x reference)
import jax
import jax.numpy as jnp
from jax.experimental import pallas as pl


def kernel(x, h, J, edge_idx_i, edge_idx_j):
    raise NotImplementedError("write your pallas kernel here")


# speedup vs baseline: 39.4160x; 39.4160x over previous
import functools

import jax
import jax.numpy as jnp
from jax import lax
from jax.experimental import pallas as pl
from jax.experimental.pallas import tpu as pltpu
from jax.experimental.pallas import tpu_sc as plsc

N_NODES = 100000
N_EDGES = 3200000
B = 16

NC = 2
NS = 16
NW = NC * NS

EPT = N_EDGES // NW
CH = 1024
NFULL = EPT // CH
TAIL = EPT - NFULL * CH
G = 128

NPT = 3136
NPAD = NW * NPT
NCH_FULL = NPT // CH
NTAIL = NPT - NCH_FULL * CH

UNROLL = 8


def _sc_body(xt_hbm, hp_hbm, j_hbm, ei_hbm, ej_hbm, out_hbm,
             ei_v, ej_v, w_v, rows_a, rows_b, acc_v, red_v, shared, gsem):
    c = lax.axis_index("c")
    s = lax.axis_index("s")
    wid = c * NS + s

    zero = jnp.zeros((B,), jnp.float32)
    accs = (zero,) * UNROLL

    ebase = wid * EPT

    def gather_pair(n_rows):
        cps = []
        for g0 in range(0, n_rows, G):
            gl = min(G, n_rows - g0)
            sl = pl.ds(g0, gl)
            cps.append(pltpu.async_copy(xt_hbm.at[ei_v.at[sl]], rows_a.at[sl], gsem))
            cps.append(pltpu.async_copy(xt_hbm.at[ej_v.at[sl]], rows_b.at[sl], gsem))
        for cp in cps:
            cp.wait()

    def edge_accum(n_rows, accs):
        def ib(i, accs):
            base = i * B
            wv = w_v[pl.ds(base, B)]
            out = list(accs)
            for j in range(B):
                k = base + j
                out[j % UNROLL] = out[j % UNROLL] + rows_a[k] * rows_b[k] * wv[j]
            return tuple(out)
        return lax.fori_loop(0, n_rows // B, ib, accs)

    def edge_chunk(base, n_rows, accs):
        sl = pl.ds(0, n_rows)
        pltpu.sync_copy(ei_hbm.at[pl.ds(base, n_rows)], ei_v.at[sl])
        pltpu.sync_copy(ej_hbm.at[pl.ds(base, n_rows)], ej_v.at[sl])
        pltpu.sync_copy(j_hbm.at[pl.ds(base, n_rows)], w_v.at[sl])
        gather_pair(n_rows)
        return edge_accum(n_rows, accs)

    def chunk_body(ch, accs):
        return edge_chunk(ebase + ch * CH, CH, accs)

    accs = lax.fori_loop(0, NFULL, chunk_body, accs)
    accs = edge_chunk(ebase + NFULL * CH, TAIL, accs)

    nbase = wid * NPT

    def node_accum(n_rows, accs):
        def ib(i, accs):
            base = i * B
            wv = w_v[pl.ds(base, B)]
            out = list(accs)
            for j in range(B):
                k = base + j
                out[j % UNROLL] = out[j % UNROLL] + rows_a[k] * wv[j]
            return tuple(out)
        return lax.fori_loop(0, n_rows // B, ib, accs)

    def node_chunk(base, n_rows, accs):
        sl = pl.ds(0, n_rows)
        pltpu.sync_copy(xt_hbm.at[pl.ds(base, n_rows)], rows_a.at[sl])
        pltpu.sync_copy(hp_hbm.at[pl.ds(base, n_rows)], w_v.at[sl])
        return node_accum(n_rows, accs)

    def nchunk_body(ch, accs):
        return node_chunk(nbase + ch * CH, CH, accs)

    accs = lax.fori_loop(0, NCH_FULL, nchunk_body, accs)
    accs = node_chunk(nbase + NCH_FULL * CH, NTAIL, accs)

    total = accs[0]
    for j in range(1, UNROLL):
        total = total + accs[j]
    acc_v[...] = total

    pltpu.sync_copy(acc_v, shared.at[s])
    plsc.subcore_barrier()

    @pl.when(s == 0)
    def _():
        pltpu.sync_copy(shared, red_v)
        t = red_v[0]
        for i in range(1, NS):
            t = t + red_v[i]
        acc_v[...] = t
        pltpu.sync_copy(acc_v, out_hbm.at[c])


@jax.jit
def kernel(x, h, J, edge_idx_i, edge_idx_j):
    xt = jnp.zeros((NPAD, B), jnp.float32).at[:N_NODES].set(x.T)
    hp = jnp.zeros((NPAD,), jnp.float32).at[:N_NODES].set(h)

    mesh = plsc.VectorSubcoreMesh(core_axis_name="c", subcore_axis_name="s")
    out2 = pl.kernel(
        _sc_body,
        out_type=jax.ShapeDtypeStruct((NC, B), jnp.float32),
        mesh=mesh,
        compiler_params=pltpu.CompilerParams(use_tc_tiling_on_sc=False),
        scratch_types=[
            pltpu.VMEM((CH,), jnp.int32),
            pltpu.VMEM((CH,), jnp.int32),
            pltpu.VMEM((CH,), jnp.float32),
            pltpu.VMEM((CH, B), jnp.float32),
            pltpu.VMEM((CH, B), jnp.float32),
            pltpu.VMEM((B,), jnp.float32),
            pltpu.VMEM((NS, B), jnp.float32),
            pltpu.VMEM_SHARED((NS, B), jnp.float32),
            pltpu.SemaphoreType.DMA,
        ],
    )(xt, hp, J, edge_idx_i, edge_idx_j)
    return out2[0] + out2[1]


# speedup vs baseline: 73.2735x; 1.8590x over previous
import functools

import jax
import jax.numpy as jnp
from jax import lax
from jax.experimental import pallas as pl
from jax.experimental.pallas import tpu as pltpu
from jax.experimental.pallas import tpu_sc as plsc

N_NODES = 100000
N_EDGES = 3200000
B = 16

NC = 2
NS = 16
NW = NC * NS

EPT = N_EDGES // NW
CH = 1024
NFULL = EPT // CH
TAIL = EPT - NFULL * CH
G = 128
NPAIR = (NFULL - 1) // 2

NPT = 3136
NPAD = NW * NPT
NCH_FULL = NPT // CH
NTAIL = NPT - NCH_FULL * CH

UNROLL = 8


def _sc_body(xt_hbm, hp_hbm, j_hbm, ei_hbm, ej_hbm, out_hbm,
             ei_v0, ei_v1, ej_v0, ej_v1, w_v0, w_v1,
             rows_a0, rows_a1, rows_b0, rows_b1,
             acc_v, red_v, shared, isem0, isem1, wsem0, wsem1, gsem0, gsem1):
    c = lax.axis_index("c")
    s = lax.axis_index("s")
    wid = c * NS + s

    ei_v = [ei_v0, ei_v1]
    ej_v = [ej_v0, ej_v1]
    w_v = [w_v0, w_v1]
    rows_a = [rows_a0, rows_a1]
    rows_b = [rows_b0, rows_b1]
    isem = [isem0, isem1]
    wsem = [wsem0, wsem1]
    gsem = [gsem0, gsem1]

    zero = jnp.zeros((B,), jnp.float32)
    accs = (zero,) * UNROLL

    ebase = wid * EPT

    def idx_copies(ch, slot):
        base = ebase + ch * CH
        return [
            pltpu.make_async_copy(ei_hbm.at[pl.ds(base, CH)], ei_v[slot], isem[slot]),
            pltpu.make_async_copy(ej_hbm.at[pl.ds(base, CH)], ej_v[slot], isem[slot]),
        ]

    def w_copy(ch, slot):
        base = ebase + ch * CH
        return pltpu.make_async_copy(j_hbm.at[pl.ds(base, CH)], w_v[slot], wsem[slot])

    def issue_idx(ch, slot):
        for cp in idx_copies(ch, slot):
            cp.start()

    def wait_idx(ch, slot):
        for cp in idx_copies(ch, slot):
            cp.wait()

    def issue_w(ch, slot):
        w_copy(ch, slot).start()

    def wait_w(ch, slot):
        w_copy(ch, slot).wait()

    def gather_copies(slot):
        cps = []
        for g0 in range(0, CH, G):
            sl = pl.ds(g0, G)
            cps.append(pltpu.make_async_copy(
                xt_hbm.at[ei_v[slot].at[sl]], rows_a[slot].at[sl], gsem[slot]))
            cps.append(pltpu.make_async_copy(
                xt_hbm.at[ej_v[slot].at[sl]], rows_b[slot].at[sl], gsem[slot]))
        return cps

    def issue_gathers(slot):
        for cp in gather_copies(slot):
            cp.start()

    def wait_gathers(slot):
        for cp in gather_copies(slot):
            cp.wait()

    def edge_accum(slot, n_rows, accs):
        ra, rb, wr = rows_a[slot], rows_b[slot], w_v[slot]

        def ib(i, accs):
            base = i * B
            wv = wr[pl.ds(base, B)]
            out = list(accs)
            for j in range(B):
                k = base + j
                out[j % UNROLL] = out[j % UNROLL] + ra[k] * rb[k] * wv[j]
            return tuple(out)
        return lax.fori_loop(0, n_rows // B, ib, accs)

    issue_idx(0, 0)
    issue_w(0, 0)
    issue_idx(1, 1)
    issue_w(1, 1)
    wait_idx(0, 0)
    issue_gathers(0)

    def pair_body(p, accs):
        c0 = p * 2
        wait_idx(c0 + 1, 1)
        issue_gathers(1)
        wait_gathers(0)
        issue_idx(c0 + 2, 0)
        wait_w(c0, 0)
        accs = edge_accum(0, CH, accs)
        issue_w(c0 + 2, 0)
        wait_idx(c0 + 2, 0)
        issue_gathers(0)
        wait_gathers(1)
        wait_w(c0 + 1, 1)
        accs = edge_accum(1, CH, accs)

        @pl.when(c0 + 3 < NFULL)
        def _():
            issue_idx(c0 + 3, 1)
            issue_w(c0 + 3, 1)
        return accs

    accs = lax.fori_loop(0, NPAIR, pair_body, accs)

    wait_gathers(0)
    wait_w(NFULL - 1, 0)
    accs = edge_accum(0, CH, accs)

    tbase = ebase + NFULL * CH
    tsl = pl.ds(0, TAIL)
    pltpu.sync_copy(ei_hbm.at[pl.ds(tbase, TAIL)], ei_v[1].at[tsl])
    pltpu.sync_copy(ej_hbm.at[pl.ds(tbase, TAIL)], ej_v[1].at[tsl])
    pltpu.sync_copy(j_hbm.at[pl.ds(tbase, TAIL)], w_v[1].at[tsl])
    tcps = []
    for g0 in range(0, TAIL, G):
        gl = min(G, TAIL - g0)
        sl = pl.ds(g0, gl)
        tcps.append(pltpu.make_async_copy(
            xt_hbm.at[ei_v[1].at[sl]], rows_a[1].at[sl], gsem[1]))
        tcps.append(pltpu.make_async_copy(
            xt_hbm.at[ej_v[1].at[sl]], rows_b[1].at[sl], gsem[1]))
    for cp in tcps:
        cp.start()
    for cp in tcps:
        cp.wait()
    accs = edge_accum(1, TAIL, accs)

    nbase = wid * NPT

    def node_accum(n_rows, accs):
        def ib(i, accs):
            base = i * B
            wv = w_v0[pl.ds(base, B)]
            out = list(accs)
            for j in range(B):
                k = base + j
                out[j % UNROLL] = out[j % UNROLL] + rows_a0[k] * wv[j]
            return tuple(out)
        return lax.fori_loop(0, n_rows // B, ib, accs)

    def node_chunk(base, n_rows, accs):
        sl = pl.ds(0, n_rows)
        pltpu.sync_copy(xt_hbm.at[pl.ds(base, n_rows)], rows_a0.at[sl])
        pltpu.sync_copy(hp_hbm.at[pl.ds(base, n_rows)], w_v0.at[sl])
        return node_accum(n_rows, accs)

    def nchunk_body(ch, accs):
        return node_chunk(nbase + ch * CH, CH, accs)

    accs = lax.fori_loop(0, NCH_FULL, nchunk_body, accs)
    accs = node_chunk(nbase + NCH_FULL * CH, NTAIL, accs)

    total = accs[0]
    for j in range(1, UNROLL):
        total = total + accs[j]
    acc_v[...] = total

    pltpu.sync_copy(acc_v, shared.at[s])
    plsc.subcore_barrier()

    @pl.when(s == 0)
    def _():
        pltpu.sync_copy(shared, red_v)
        t = red_v[0]
        for i in range(1, NS):
            t = t + red_v[i]
        acc_v[...] = t
        pltpu.sync_copy(acc_v, out_hbm.at[c])


@jax.jit
def kernel(x, h, J, edge_idx_i, edge_idx_j):
    xt = jnp.zeros((NPAD, B), jnp.float32).at[:N_NODES].set(x.T)
    hp = jnp.zeros((NPAD,), jnp.float32).at[:N_NODES].set(h)

    mesh = plsc.VectorSubcoreMesh(core_axis_name="c", subcore_axis_name="s")
    out2 = pl.kernel(
        _sc_body,
        out_type=jax.ShapeDtypeStruct((NC, B), jnp.float32),
        mesh=mesh,
        compiler_params=pltpu.CompilerParams(use_tc_tiling_on_sc=False),
        scratch_types=[
            pltpu.VMEM((CH,), jnp.int32),
            pltpu.VMEM((CH,), jnp.int32),
            pltpu.VMEM((CH,), jnp.int32),
            pltpu.VMEM((CH,), jnp.int32),
            pltpu.VMEM((CH,), jnp.float32),
            pltpu.VMEM((CH,), jnp.float32),
            pltpu.VMEM((CH, B), jnp.float32),
            pltpu.VMEM((CH, B), jnp.float32),
            pltpu.VMEM((CH, B), jnp.float32),
            pltpu.VMEM((CH, B), jnp.float32),
            pltpu.VMEM((B,), jnp.float32),
            pltpu.VMEM((NS, B), jnp.float32),
            pltpu.VMEM_SHARED((NS, B), jnp.float32),
            pltpu.SemaphoreType.DMA,
            pltpu.SemaphoreType.DMA,
            pltpu.SemaphoreType.DMA,
            pltpu.SemaphoreType.DMA,
            pltpu.SemaphoreType.DMA,
            pltpu.SemaphoreType.DMA,
        ],
    )(xt, hp, J, edge_idx_i, edge_idx_j)
    return out2[0] + out2[1]


# speedup vs baseline: 81.8308x; 1.1168x over previous
import functools

import jax
import jax.numpy as jnp
from jax import lax
from jax.experimental import pallas as pl
from jax.experimental.pallas import tpu as pltpu
from jax.experimental.pallas import tpu_sc as plsc

N_NODES = 100000
N_EDGES = 3200000
B = 16

NC = 2
NS = 16
NW = NC * NS

EPT = N_EDGES // NW
CH = 1024
NFULL = EPT // CH
TAIL = EPT - NFULL * CH
G = 128
NPAIR = (NFULL - 1) // 2

UNROLL = 8


def _sc_body(xt_hbm, j_hbm, ei_hbm, ej_hbm, out_hbm,
             ei_v0, ei_v1, ej_v0, ej_v1, w_v0, w_v1,
             rows_a0, rows_a1, rows_b0, rows_b1,
             acc_v, red_v, shared, isem0, isem1, wsem0, wsem1, gsem0, gsem1):
    c = lax.axis_index("c")
    s = lax.axis_index("s")
    wid = c * NS + s

    ei_v = [ei_v0, ei_v1]
    ej_v = [ej_v0, ej_v1]
    w_v = [w_v0, w_v1]
    rows_a = [rows_a0, rows_a1]
    rows_b = [rows_b0, rows_b1]
    isem = [isem0, isem1]
    wsem = [wsem0, wsem1]
    gsem = [gsem0, gsem1]

    zero = jnp.zeros((B,), jnp.float32)
    accs = (zero,) * UNROLL

    ebase = wid * EPT

    def idx_copies(ch, slot):
        base = ebase + ch * CH
        return [
            pltpu.make_async_copy(ei_hbm.at[pl.ds(base, CH)], ei_v[slot], isem[slot]),
            pltpu.make_async_copy(ej_hbm.at[pl.ds(base, CH)], ej_v[slot], isem[slot]),
        ]

    def w_copy(ch, slot):
        base = ebase + ch * CH
        return pltpu.make_async_copy(j_hbm.at[pl.ds(base, CH)], w_v[slot], wsem[slot])

    def issue_idx(ch, slot):
        for cp in idx_copies(ch, slot):
            cp.start()

    def wait_idx(ch, slot):
        for cp in idx_copies(ch, slot):
            cp.wait()

    def issue_w(ch, slot):
        w_copy(ch, slot).start()

    def wait_w(ch, slot):
        w_copy(ch, slot).wait()

    def gather_copies(slot):
        cps = []
        for g0 in range(0, CH, G):
            sl = pl.ds(g0, G)
            cps.append(pltpu.make_async_copy(
                xt_hbm.at[ei_v[slot].at[sl]], rows_a[slot].at[sl], gsem[slot]))
            cps.append(pltpu.make_async_copy(
                xt_hbm.at[ej_v[slot].at[sl]], rows_b[slot].at[sl], gsem[slot]))
        return cps

    def issue_gathers(slot):
        for cp in gather_copies(slot):
            cp.start()

    def wait_gathers(slot):
        for cp in gather_copies(slot):
            cp.wait()

    def edge_accum(slot, n_rows, accs):
        ra, rb, wr = rows_a[slot], rows_b[slot], w_v[slot]

        def ib(i, accs):
            base = i * B
            wv = wr[pl.ds(base, B)]
            out = list(accs)
            for j in range(B):
                k = base + j
                out[j % UNROLL] = out[j % UNROLL] + ra[k] * rb[k] * wv[j]
            return tuple(out)
        return lax.fori_loop(0, n_rows // B, ib, accs)

    issue_idx(0, 0)
    issue_w(0, 0)
    issue_idx(1, 1)
    issue_w(1, 1)
    wait_idx(0, 0)
    issue_gathers(0)

    def pair_body(p, accs):
        c0 = p * 2
        wait_idx(c0 + 1, 1)
        issue_gathers(1)
        wait_gathers(0)
        issue_idx(c0 + 2, 0)
        wait_w(c0, 0)
        accs = edge_accum(0, CH, accs)
        issue_w(c0 + 2, 0)
        wait_idx(c0 + 2, 0)
        issue_gathers(0)
        wait_gathers(1)
        wait_w(c0 + 1, 1)
        accs = edge_accum(1, CH, accs)

        @pl.when(c0 + 3 < NFULL)
        def _():
            issue_idx(c0 + 3, 1)
            issue_w(c0 + 3, 1)
        return accs

    accs = lax.fori_loop(0, NPAIR, pair_body, accs)

    wait_gathers(0)
    wait_w(NFULL - 1, 0)
    accs = edge_accum(0, CH, accs)

    tbase = ebase + NFULL * CH
    tsl = pl.ds(0, TAIL)
    pltpu.sync_copy(ei_hbm.at[pl.ds(tbase, TAIL)], ei_v[1].at[tsl])
    pltpu.sync_copy(ej_hbm.at[pl.ds(tbase, TAIL)], ej_v[1].at[tsl])
    pltpu.sync_copy(j_hbm.at[pl.ds(tbase, TAIL)], w_v[1].at[tsl])
    tcps = []
    for g0 in range(0, TAIL, G):
        gl = min(G, TAIL - g0)
        sl = pl.ds(g0, gl)
        tcps.append(pltpu.make_async_copy(
            xt_hbm.at[ei_v[1].at[sl]], rows_a[1].at[sl], gsem[1]))
        tcps.append(pltpu.make_async_copy(
            xt_hbm.at[ej_v[1].at[sl]], rows_b[1].at[sl], gsem[1]))
    for cp in tcps:
        cp.start()
    for cp in tcps:
        cp.wait()
    accs = edge_accum(1, TAIL, accs)

    total = accs[0]
    for j in range(1, UNROLL):
        total = total + accs[j]
    acc_v[...] = total

    pltpu.sync_copy(acc_v, shared.at[s])
    plsc.subcore_barrier()

    @pl.when(s == 0)
    def _():
        pltpu.sync_copy(shared, red_v)
        t = red_v[0]
        for i in range(1, NS):
            t = t + red_v[i]
        acc_v[...] = t
        pltpu.sync_copy(acc_v, out_hbm.at[c])


def _hdot_body(x_ref, h_ref, out_ref):
    out_ref[...] = jnp.sum(x_ref[...] * h_ref[...][None, :], axis=1)


@jax.jit
def kernel(x, h, J, edge_idx_i, edge_idx_j):
    xt_flat = x.T.reshape(-1)
    xtr = xt_flat.reshape(N_NODES, B)

    mesh = plsc.VectorSubcoreMesh(core_axis_name="c", subcore_axis_name="s")
    out2 = pl.kernel(
        _sc_body,
        out_type=jax.ShapeDtypeStruct((NC, B), jnp.float32),
        mesh=mesh,
        compiler_params=pltpu.CompilerParams(use_tc_tiling_on_sc=False),
        scratch_types=[
            pltpu.VMEM((CH,), jnp.int32),
            pltpu.VMEM((CH,), jnp.int32),
            pltpu.VMEM((CH,), jnp.int32),
            pltpu.VMEM((CH,), jnp.int32),
            pltpu.VMEM((CH,), jnp.float32),
            pltpu.VMEM((CH,), jnp.float32),
            pltpu.VMEM((CH, B), jnp.float32),
            pltpu.VMEM((CH, B), jnp.float32),
            pltpu.VMEM((CH, B), jnp.float32),
            pltpu.VMEM((CH, B), jnp.float32),
            pltpu.VMEM((B,), jnp.float32),
            pltpu.VMEM((NS, B), jnp.float32),
            pltpu.VMEM_SHARED((NS, B), jnp.float32),
            pltpu.SemaphoreType.DMA,
            pltpu.SemaphoreType.DMA,
            pltpu.SemaphoreType.DMA,
            pltpu.SemaphoreType.DMA,
            pltpu.SemaphoreType.DMA,
            pltpu.SemaphoreType.DMA,
        ],
    )(xtr, J, edge_idx_i, edge_idx_j)

    hdot = pl.pallas_call(
        _hdot_body,
        out_shape=jax.ShapeDtypeStruct((B,), jnp.float32),
    )(x, h)

    return out2[0] + out2[1] + hdot


# speedup vs baseline: 84.9753x; 1.0384x over previous
import functools

import jax
import jax.numpy as jnp
from jax import lax
from jax.experimental import pallas as pl
from jax.experimental.pallas import tpu as pltpu
from jax.experimental.pallas import tpu_sc as plsc

N_NODES = 100000
N_EDGES = 3200000
B = 16

NC = 2
NS = 16
NW = NC * NS

EPT = N_EDGES // NW
CH = 1536
NFULL = EPT // CH
TAIL = EPT - NFULL * CH
G = 128
NPAIR = (NFULL - 1) // 2

UNROLL = 8


def _sc_body(xt_hbm, j_hbm, ei_hbm, ej_hbm, out_hbm,
             ei_v0, ei_v1, ej_v0, ej_v1, w_v0, w_v1,
             rows_a0, rows_a1, rows_b0, rows_b1,
             acc_v, red_v, shared, isem0, isem1, wsem0, wsem1, gsem0, gsem1):
    c = lax.axis_index("c")
    s = lax.axis_index("s")
    wid = c * NS + s

    ei_v = [ei_v0, ei_v1]
    ej_v = [ej_v0, ej_v1]
    w_v = [w_v0, w_v1]
    rows_a = [rows_a0, rows_a1]
    rows_b = [rows_b0, rows_b1]
    isem = [isem0, isem1]
    wsem = [wsem0, wsem1]
    gsem = [gsem0, gsem1]

    zero = jnp.zeros((B,), jnp.float32)
    accs = (zero,) * UNROLL

    ebase = wid * EPT

    def idx_copies(ch, slot):
        base = ebase + ch * CH
        return [
            pltpu.make_async_copy(ei_hbm.at[pl.ds(base, CH)], ei_v[slot], isem[slot]),
            pltpu.make_async_copy(ej_hbm.at[pl.ds(base, CH)], ej_v[slot], isem[slot]),
        ]

    def w_copy(ch, slot):
        base = ebase + ch * CH
        return pltpu.make_async_copy(j_hbm.at[pl.ds(base, CH)], w_v[slot], wsem[slot])

    def issue_idx(ch, slot):
        for cp in idx_copies(ch, slot):
            cp.start()

    def wait_idx(ch, slot):
        for cp in idx_copies(ch, slot):
            cp.wait()

    def issue_w(ch, slot):
        w_copy(ch, slot).start()

    def wait_w(ch, slot):
        w_copy(ch, slot).wait()

    def gather_copies(slot):
        cps = []
        for g0 in range(0, CH, G):
            sl = pl.ds(g0, G)
            cps.append(pltpu.make_async_copy(
                xt_hbm.at[ei_v[slot].at[sl]], rows_a[slot].at[sl], gsem[slot]))
            cps.append(pltpu.make_async_copy(
                xt_hbm.at[ej_v[slot].at[sl]], rows_b[slot].at[sl], gsem[slot]))
        return cps

    def issue_gathers(slot):
        for cp in gather_copies(slot):
            cp.start()

    def wait_gathers(slot):
        for cp in gather_copies(slot):
            cp.wait()

    def edge_accum(slot, n_rows, accs):
        ra, rb, wr = rows_a[slot], rows_b[slot], w_v[slot]

        def ib(i, accs):
            base = i * B
            wv = wr[pl.ds(base, B)]
            out = list(accs)
            for j in range(B):
                k = base + j
                out[j % UNROLL] = out[j % UNROLL] + ra[k] * rb[k] * wv[j]
            return tuple(out)
        return lax.fori_loop(0, n_rows // B, ib, accs)

    issue_idx(0, 0)
    issue_w(0, 0)
    issue_idx(1, 1)
    issue_w(1, 1)
    wait_idx(0, 0)
    issue_gathers(0)

    def pair_body(p, accs):
        c0 = p * 2
        wait_idx(c0 + 1, 1)
        issue_gathers(1)
        wait_gathers(0)
        issue_idx(c0 + 2, 0)
        wait_w(c0, 0)
        accs = edge_accum(0, CH, accs)
        issue_w(c0 + 2, 0)
        wait_idx(c0 + 2, 0)
        issue_gathers(0)
        wait_gathers(1)
        wait_w(c0 + 1, 1)
        accs = edge_accum(1, CH, accs)

        @pl.when(c0 + 3 < NFULL)
        def _():
            issue_idx(c0 + 3, 1)
            issue_w(c0 + 3, 1)
        return accs

    accs = lax.fori_loop(0, NPAIR, pair_body, accs)

    wait_gathers(0)
    wait_w(NFULL - 1, 0)
    accs = edge_accum(0, CH, accs)

    tbase = ebase + NFULL * CH
    tsl = pl.ds(0, TAIL)
    pltpu.sync_copy(ei_hbm.at[pl.ds(tbase, TAIL)], ei_v[1].at[tsl])
    pltpu.sync_copy(ej_hbm.at[pl.ds(tbase, TAIL)], ej_v[1].at[tsl])
    pltpu.sync_copy(j_hbm.at[pl.ds(tbase, TAIL)], w_v[1].at[tsl])
    tcps = []
    for g0 in range(0, TAIL, G):
        gl = min(G, TAIL - g0)
        sl = pl.ds(g0, gl)
        tcps.append(pltpu.make_async_copy(
            xt_hbm.at[ei_v[1].at[sl]], rows_a[1].at[sl], gsem[1]))
        tcps.append(pltpu.make_async_copy(
            xt_hbm.at[ej_v[1].at[sl]], rows_b[1].at[sl], gsem[1]))
    for cp in tcps:
        cp.start()
    for cp in tcps:
        cp.wait()
    accs = edge_accum(1, TAIL, accs)

    total = accs[0]
    for j in range(1, UNROLL):
        total = total + accs[j]
    acc_v[...] = total

    pltpu.sync_copy(acc_v, shared.at[s])
    plsc.subcore_barrier()

    @pl.when(s == 0)
    def _():
        pltpu.sync_copy(shared, red_v)
        t = red_v[0]
        for i in range(1, NS):
            t = t + red_v[i]
        acc_v[...] = t
        pltpu.sync_copy(acc_v, out_hbm.at[c])


def _hdot_body(x_ref, h_ref, out_ref):
    out_ref[...] = jnp.sum(x_ref[...] * h_ref[...][None, :], axis=1)


@jax.jit
def kernel(x, h, J, edge_idx_i, edge_idx_j):
    xt_flat = x.T.reshape(-1)
    xtr = xt_flat.reshape(N_NODES, B)

    mesh = plsc.VectorSubcoreMesh(core_axis_name="c", subcore_axis_name="s")
    out2 = pl.kernel(
        _sc_body,
        out_type=jax.ShapeDtypeStruct((NC, B), jnp.float32),
        mesh=mesh,
        compiler_params=pltpu.CompilerParams(use_tc_tiling_on_sc=False),
        scratch_types=[
            pltpu.VMEM((CH,), jnp.int32),
            pltpu.VMEM((CH,), jnp.int32),
            pltpu.VMEM((CH,), jnp.int32),
            pltpu.VMEM((CH,), jnp.int32),
            pltpu.VMEM((CH,), jnp.float32),
            pltpu.VMEM((CH,), jnp.float32),
            pltpu.VMEM((CH, B), jnp.float32),
            pltpu.VMEM((CH, B), jnp.float32),
            pltpu.VMEM((CH, B), jnp.float32),
            pltpu.VMEM((CH, B), jnp.float32),
            pltpu.VMEM((B,), jnp.float32),
            pltpu.VMEM((NS, B), jnp.float32),
            pltpu.VMEM_SHARED((NS, B), jnp.float32),
            pltpu.SemaphoreType.DMA,
            pltpu.SemaphoreType.DMA,
            pltpu.SemaphoreType.DMA,
            pltpu.SemaphoreType.DMA,
            pltpu.SemaphoreType.DMA,
            pltpu.SemaphoreType.DMA,
        ],
    )(xtr, J, edge_idx_i, edge_idx_j)

    hdot = pl.pallas_call(
        _hdot_body,
        out_shape=jax.ShapeDtypeStruct((B,), jnp.float32),
    )(x, h)

    return out2[0] + out2[1] + hdot


# speedup vs baseline: 85.6401x; 1.0078x over previous
import functools

import jax
import jax.numpy as jnp
from jax import lax
from jax.experimental import pallas as pl
from jax.experimental.pallas import tpu as pltpu
from jax.experimental.pallas import tpu_sc as plsc

N_NODES = 100000
N_EDGES = 3200000
B = 16

NC = 2
NS = 16
NW = NC * NS

EPT = N_EDGES // NW
CH = 1792
NFULL = EPT // CH
TAIL = EPT - NFULL * CH
G = 128
NPAIR = (NFULL - 1) // 2

UNROLL = 8


def _sc_body(xt_hbm, j_hbm, ei_hbm, ej_hbm, out_hbm,
             ei_v0, ei_v1, ej_v0, ej_v1, w_v0, w_v1,
             rows_a0, rows_a1, rows_b0, rows_b1,
             acc_v, red_v, shared, isem0, isem1, wsem0, wsem1, gsem0, gsem1):
    c = lax.axis_index("c")
    s = lax.axis_index("s")
    wid = c * NS + s

    ei_v = [ei_v0, ei_v1]
    ej_v = [ej_v0, ej_v1]
    w_v = [w_v0, w_v1]
    rows_a = [rows_a0, rows_a1]
    rows_b = [rows_b0, rows_b1]
    isem = [isem0, isem1]
    wsem = [wsem0, wsem1]
    gsem = [gsem0, gsem1]

    zero = jnp.zeros((B,), jnp.float32)
    accs = (zero,) * UNROLL

    ebase = wid * EPT

    def idx_copies(ch, slot):
        base = ebase + ch * CH
        return [
            pltpu.make_async_copy(ei_hbm.at[pl.ds(base, CH)], ei_v[slot], isem[slot]),
            pltpu.make_async_copy(ej_hbm.at[pl.ds(base, CH)], ej_v[slot], isem[slot]),
        ]

    def w_copy(ch, slot):
        base = ebase + ch * CH
        return pltpu.make_async_copy(j_hbm.at[pl.ds(base, CH)], w_v[slot], wsem[slot])

    def issue_idx(ch, slot):
        for cp in idx_copies(ch, slot):
            cp.start()

    def wait_idx(ch, slot):
        for cp in idx_copies(ch, slot):
            cp.wait()

    def issue_w(ch, slot):
        w_copy(ch, slot).start()

    def wait_w(ch, slot):
        w_copy(ch, slot).wait()

    def gather_copies(slot):
        cps = []
        for g0 in range(0, CH, G):
            sl = pl.ds(g0, G)
            cps.append(pltpu.make_async_copy(
                xt_hbm.at[ei_v[slot].at[sl]], rows_a[slot].at[sl], gsem[slot]))
            cps.append(pltpu.make_async_copy(
                xt_hbm.at[ej_v[slot].at[sl]], rows_b[slot].at[sl], gsem[slot]))
        return cps

    def issue_gathers(slot):
        for cp in gather_copies(slot):
            cp.start()

    def wait_gathers(slot):
        for cp in gather_copies(slot):
            cp.wait()

    def edge_accum(slot, n_rows, accs):
        ra, rb, wr = rows_a[slot], rows_b[slot], w_v[slot]

        def ib(i, accs):
            base = i * B
            wv = wr[pl.ds(base, B)]
            out = list(accs)
            for j in range(B):
                k = base + j
                out[j % UNROLL] = out[j % UNROLL] + ra[k] * rb[k] * wv[j]
            return tuple(out)
        return lax.fori_loop(0, n_rows // B, ib, accs)

    issue_idx(0, 0)
    issue_w(0, 0)
    issue_idx(1, 1)
    issue_w(1, 1)
    wait_idx(0, 0)
    issue_gathers(0)

    def pair_body(p, accs):
        c0 = p * 2
        wait_idx(c0 + 1, 1)
        issue_gathers(1)
        wait_gathers(0)
        issue_idx(c0 + 2, 0)
        wait_w(c0, 0)
        accs = edge_accum(0, CH, accs)
        issue_w(c0 + 2, 0)
        wait_idx(c0 + 2, 0)
        issue_gathers(0)
        wait_gathers(1)
        wait_w(c0 + 1, 1)
        accs = edge_accum(1, CH, accs)

        @pl.when(c0 + 3 < NFULL)
        def _():
            issue_idx(c0 + 3, 1)
            issue_w(c0 + 3, 1)
        return accs

    accs = lax.fori_loop(0, NPAIR, pair_body, accs)

    wait_gathers(0)
    wait_w(NFULL - 1, 0)
    accs = edge_accum(0, CH, accs)

    tbase = ebase + NFULL * CH
    tsl = pl.ds(0, TAIL)
    pltpu.sync_copy(ei_hbm.at[pl.ds(tbase, TAIL)], ei_v[1].at[tsl])
    pltpu.sync_copy(ej_hbm.at[pl.ds(tbase, TAIL)], ej_v[1].at[tsl])
    pltpu.sync_copy(j_hbm.at[pl.ds(tbase, TAIL)], w_v[1].at[tsl])
    tcps = []
    for g0 in range(0, TAIL, G):
        gl = min(G, TAIL - g0)
        sl = pl.ds(g0, gl)
        tcps.append(pltpu.make_async_copy(
            xt_hbm.at[ei_v[1].at[sl]], rows_a[1].at[sl], gsem[1]))
        tcps.append(pltpu.make_async_copy(
            xt_hbm.at[ej_v[1].at[sl]], rows_b[1].at[sl], gsem[1]))
    for cp in tcps:
        cp.start()
    for cp in tcps:
        cp.wait()
    accs = edge_accum(1, TAIL, accs)

    total = accs[0]
    for j in range(1, UNROLL):
        total = total + accs[j]
    acc_v[...] = total

    pltpu.sync_copy(acc_v, shared.at[s])
    plsc.subcore_barrier()

    @pl.when(s == 0)
    def _():
        pltpu.sync_copy(shared, red_v)
        t = red_v[0]
        for i in range(1, NS):
            t = t + red_v[i]
        acc_v[...] = t
        pltpu.sync_copy(acc_v, out_hbm.at[c])


def _hdot_body(x_ref, h_ref, out_ref):
    out_ref[...] = jnp.sum(x_ref[...] * h_ref[...][None, :], axis=1)


@jax.jit
def kernel(x, h, J, edge_idx_i, edge_idx_j):
    xt_flat = x.T.reshape(-1)
    xtr = xt_flat.reshape(N_NODES, B)

    mesh = plsc.VectorSubcoreMesh(core_axis_name="c", subcore_axis_name="s")
    out2 = pl.kernel(
        _sc_body,
        out_type=jax.ShapeDtypeStruct((NC, B), jnp.float32),
        mesh=mesh,
        compiler_params=pltpu.CompilerParams(use_tc_tiling_on_sc=False),
        scratch_types=[
            pltpu.VMEM((CH,), jnp.int32),
            pltpu.VMEM((CH,), jnp.int32),
            pltpu.VMEM((CH,), jnp.int32),
            pltpu.VMEM((CH,), jnp.int32),
            pltpu.VMEM((CH,), jnp.float32),
            pltpu.VMEM((CH,), jnp.float32),
            pltpu.VMEM((CH, B), jnp.float32),
            pltpu.VMEM((CH, B), jnp.float32),
            pltpu.VMEM((CH, B), jnp.float32),
            pltpu.VMEM((CH, B), jnp.float32),
            pltpu.VMEM((B,), jnp.float32),
            pltpu.VMEM((NS, B), jnp.float32),
            pltpu.VMEM_SHARED((NS, B), jnp.float32),
            pltpu.SemaphoreType.DMA,
            pltpu.SemaphoreType.DMA,
            pltpu.SemaphoreType.DMA,
            pltpu.SemaphoreType.DMA,
            pltpu.SemaphoreType.DMA,
            pltpu.SemaphoreType.DMA,
        ],
    )(xtr, J, edge_idx_i, edge_idx_j)

    hdot = pl.pallas_call(
        _hdot_body,
        out_shape=jax.ShapeDtypeStruct((B,), jnp.float32),
    )(x, h)

    return out2[0] + out2[1] + hdot
